# Initial kernel scaffold; baseline (speedup 1.0000x reference)
import functools

import jax
import jax.numpy as jnp
from jax import lax
from jax.experimental import pallas as pl
from jax.experimental.pallas import tpu as pltpu
from jax.experimental.pallas import tpu_sc as plsc

N_PAD = 10240
BLK = 512
D = 128
NC, NS = 2, 16
NW = NC * NS
CHUNK = 128
NCHUNK = 80
E_PAD = NW * NCHUNK * CHUNK
RPT = N_PAD // NS

_mesh = plsc.VectorSubcoreMesh(core_axis_name="c", subcore_axis_name="s")


@functools.partial(
    pl.kernel,
    out_type=jax.ShapeDtypeStruct((NC, N_PAD, 16), jnp.float32),
    mesh=_mesh,
    scratch_types=[
        pltpu.VMEM((NCHUNK, CHUNK), jnp.int32),
        pltpu.VMEM((CHUNK, 16), jnp.float32),
        pltpu.VMEM_SHARED((N_PAD, 16), jnp.float32),
    ],
)
def _deg_kernel(col3_hbm, zeros16_hbm, ones_hbm, out_hbm, colv, onesv, acc):
    c = lax.axis_index("c")
    s = lax.axis_index("s")
    wid = c * NS + s
    base = s * RPT
    pltpu.sync_copy(zeros16_hbm.at[pl.ds(base, RPT)], acc.at[pl.ds(base, RPT)])
    pltpu.sync_copy(ones_hbm, onesv)
    pltpu.sync_copy(col3_hbm.at[wid], colv)
    plsc.subcore_barrier()

    def body(j, carry):
        pltpu.sync_copy(onesv, acc.at[colv.at[j]], add=True)
        return carry

    lax.fori_loop(0, NCHUNK, body, 0)
    plsc.subcore_barrier()
    pltpu.sync_copy(acc.at[pl.ds(base, RPT)], out_hbm.at[c, pl.ds(base, RPT)])


@functools.partial(
    pl.kernel,
    out_type=jax.ShapeDtypeStruct((NC, N_PAD, D), jnp.float32),
    mesh=_mesh,
    scratch_types=[
        pltpu.VMEM((NCHUNK, CHUNK), jnp.int32),
        pltpu.VMEM((NCHUNK, CHUNK), jnp.int32),
        pltpu.VMEM((2, CHUNK, D), jnp.float32),
        pltpu.VMEM_SHARED((N_PAD, D), jnp.float32),
        pltpu.SemaphoreType.DMA,
    ],
)
def _prop_kernel(src_hbm, row3_hbm, col3_hbm, zeros_hbm, out_hbm,
                 rowv, colv, buf, acc, gsem):
    c = lax.axis_index("c")
    s = lax.axis_index("s")
    wid = c * NS + s
    base = s * RPT
    pltpu.sync_copy(zeros_hbm.at[pl.ds(base, RPT)], acc.at[pl.ds(base, RPT)])
    pltpu.sync_copy(row3_hbm.at[wid], rowv)
    pltpu.sync_copy(col3_hbm.at[wid], colv)
    plsc.subcore_barrier()

    pltpu.make_async_copy(src_hbm.at[colv.at[0]], buf.at[0], gsem).start()

    def body(j, carry):
        slot = lax.rem(j, 2)
        pltpu.make_async_copy(src_hbm.at[colv.at[j]], buf.at[slot], gsem).wait()

        @pl.when(j < NCHUNK - 1)
        def _():
            pltpu.make_async_copy(
                src_hbm.at[colv.at[j + 1]], buf.at[1 - slot], gsem).start()

        pltpu.sync_copy(buf.at[slot], acc.at[rowv.at[j]], add=True)
        return carry

    lax.fori_loop(0, NCHUNK, body, 0)
    plsc.subcore_barrier()
    pltpu.sync_copy(acc.at[pl.ds(base, RPT)], out_hbm.at[c, pl.ds(base, RPT)])


def _stage1_body(x_ref, w_ref, b_ref, dp_ref, dis_ref, g_ref, gsc_ref):
    deg = dp_ref[0, :, 0:1] + dp_ref[1, :, 0:1] + 1.0
    dis = jnp.broadcast_to(lax.rsqrt(deg), (BLK, D))
    g = jnp.dot(x_ref[...], w_ref[...], preferred_element_type=jnp.float32)
    g = g + b_ref[...]
    dis_ref[...] = dis
    g_ref[...] = g
    gsc_ref[...] = dis * g


def _stage_mid_body(p_ref, g_ref, dis_ref, w_ref, b_ref, gout_ref, gsc_ref):
    dis = dis_ref[...]
    h = dis * (p_ref[0] + p_ref[1]) + dis * dis * g_ref[...]
    g = jnp.dot(h, w_ref[...], preferred_element_type=jnp.float32) + b_ref[...]
    gout_ref[...] = g
    gsc_ref[...] = dis * g


def _stage_fin_body(p_ref, g_ref, dis_ref, out_ref):
    dis = dis_ref[...]
    out_ref[...] = dis * (p_ref[0] + p_ref[1]) + dis * dis * g_ref[...]


_ROWS = pl.BlockSpec((BLK, D), lambda i: (i, 0))
_WMAT = pl.BlockSpec((D, D), lambda i: (0, 0))
_BVEC = pl.BlockSpec((1, D), lambda i: (0, 0))
_PART = pl.BlockSpec((NC, BLK, D), lambda i: (0, i, 0))
_DEGP = pl.BlockSpec((NC, BLK, 16), lambda i: (0, i, 0))
_GRID = (N_PAD // BLK,)
_F32 = functools.partial(jax.ShapeDtypeStruct, dtype=jnp.float32)


def _stage1(x_pad, w1t, b1r, degp):
    return pl.pallas_call(
        _stage1_body,
        grid=_GRID,
        in_specs=[_ROWS, _WMAT, _BVEC, _DEGP],
        out_specs=[_ROWS, _ROWS, _ROWS],
        out_shape=[_F32((N_PAD, D))] * 3,
    )(x_pad, w1t, b1r, degp)


def _stage_mid(p, g, dis, wt, br):
    return pl.pallas_call(
        _stage_mid_body,
        grid=_GRID,
        in_specs=[_PART, _ROWS, _ROWS, _WMAT, _BVEC],
        out_specs=[_ROWS, _ROWS],
        out_shape=[_F32((N_PAD, D))] * 2,
    )(p, g, dis, wt, br)


def _stage_fin(p, g, dis):
    return pl.pallas_call(
        _stage_fin_body,
        grid=_GRID,
        in_specs=[_PART, _ROWS, _ROWS],
        out_specs=_ROWS,
        out_shape=_F32((N_PAD, D)),
    )(p, g, dis)


def kernel(x, edge_index, W1, b1, W2, b2, W3, b3):
    N = x.shape[0]
    E = edge_index.shape[1]
    row = edge_index[0].astype(jnp.int32)
    col = edge_index[1].astype(jnp.int32)
    dummy = jnp.full((E_PAD - E,), N, jnp.int32)
    row3 = jnp.concatenate([row, dummy]).reshape(NW, NCHUNK, CHUNK)
    col3 = jnp.concatenate([col, dummy]).reshape(NW, NCHUNK, CHUNK)

    x_pad = jnp.zeros((N_PAD, D), jnp.float32).at[:N].set(x)
    zeros128 = jnp.zeros((N_PAD, D), jnp.float32)
    zeros16 = jnp.zeros((N_PAD, 16), jnp.float32)
    ones16 = jnp.ones((CHUNK, 16), jnp.float32)
    w1t, w2t, w3t = W1.T, W2.T, W3.T
    b1r, b2r, b3r = b1.reshape(1, D), b2.reshape(1, D), b3.reshape(1, D)

    degp = _deg_kernel(col3, zeros16, ones16)
    dis, g1, gsc1 = _stage1(x_pad, w1t, b1r, degp)
    p1 = _prop_kernel(gsc1, row3, col3, zeros128)
    g2, gsc2 = _stage_mid(p1, g1, dis, w2t, b2r)
    p2 = _prop_kernel(gsc2, row3, col3, zeros128)
    g3, gsc3 = _stage_mid(p2, g2, dis, w3t, b3r)
    p3 = _prop_kernel(gsc3, row3, col3, zeros128)
    h = _stage_fin(p3, g3, dis)
    return h[:N]


# speedup vs baseline: 5.1762x; 5.1762x over previous
import functools

import jax
import jax.numpy as jnp
from jax import lax
from jax.experimental import pallas as pl
from jax.experimental.pallas import tpu as pltpu
from jax.experimental.pallas import tpu_sc as plsc

N_PAD = 10240
BLK = 512
D = 128
NS = 16
CHUNK = 80
NCHUNK = 256
E_PAD = NS * NCHUNK * CHUNK
VCHUNK = 128
VNCHUNK = 160
RPT = N_PAD // NS
GUARD = 1024
GUARDU = 0
SC16 = 8
SC128 = 1

_mesh = plsc.VectorSubcoreMesh(
    core_axis_name="c", subcore_axis_name="s", num_cores=1)


@functools.partial(
    pl.kernel,
    out_type=jax.ShapeDtypeStruct((N_PAD, D), jnp.float32),
    mesh=_mesh,
    scratch_types=[
        pltpu.VMEM((2, CHUNK), jnp.int32),
        pltpu.VMEM((CHUNK, D), jnp.float32),
        pltpu.VMEM((CHUNK, D), jnp.float32),
        pltpu.VMEM_SHARED((N_PAD, D), jnp.float32),
        pltpu.SemaphoreType.DMA,
    ],
)
def _deg_kernel(col3_hbm, zeros_hbm, ones_hbm, out_hbm, coli, onesv, stg,
                acc, semc):
    s = lax.axis_index("s")
    base = s * RPT
    pltpu.sync_copy(ones_hbm, onesv)
    for c in range(RPT // CHUNK):
        off = base + c * CHUNK
        pltpu.sync_copy(zeros_hbm.at[pl.ds(off, CHUNK)], stg)
        pltpu.sync_copy(stg, acc.at[pl.ds(off, CHUNK)])
    plsc.subcore_barrier()

    pltpu.make_async_copy(col3_hbm.at[s, 0], coli.at[0], semc).start()
    pltpu.make_async_copy(col3_hbm.at[s, 0], coli.at[0], semc).wait()
    pltpu.make_async_copy(col3_hbm.at[s, 1], coli.at[1], semc).start()

    def body(j, carry):
        slot = lax.rem(j, 2)

        @pl.when(j < NCHUNK - 1)
        def _():
            pltpu.make_async_copy(col3_hbm.at[s, 0], coli.at[1 - slot],
                                  semc).wait()

        pltpu.sync_copy(onesv, acc.at[coli.at[slot]], add=True)

        @pl.when(j < NCHUNK - 2)
        def _():
            pltpu.make_async_copy(col3_hbm.at[s, j + 2], coli.at[slot],
                                  semc).start()

        return carry

    lax.fori_loop(0, NCHUNK, body, 0)
    plsc.subcore_barrier()
    for c in range(RPT // CHUNK):
        off = base + c * CHUNK
        pltpu.sync_copy(acc.at[pl.ds(off, CHUNK)], stg)
        pltpu.sync_copy(stg, out_hbm.at[pl.ds(off, CHUNK)])


@functools.partial(
    pl.kernel,
    out_type=[jax.ShapeDtypeStruct((N_PAD, D), jnp.float32)] * 3,
    mesh=_mesh,
    scratch_types=[
        pltpu.VMEM((2, CHUNK), jnp.int32),
        pltpu.VMEM((2, CHUNK), jnp.int32),
        pltpu.VMEM((2, CHUNK, D), jnp.float32),
        pltpu.VMEM((RPT // 8, D), jnp.float32),
        pltpu.VMEM((RPT // 8, D), jnp.float32),
        pltpu.VMEM((1, D), jnp.float32),
        pltpu.VMEM((1, D), jnp.float32),
        pltpu.VMEM_SHARED((GUARDU + N_PAD, D), jnp.float32),
        pltpu.SemaphoreType.DMA,
        pltpu.SemaphoreType.DMA,
        pltpu.SemaphoreType.DMA,
    ],
)
def _umega_kernel(u0, dis2r, disr, c2r, c3r, row3_hbm, col3_hbm,
                  u1o, u2o, u3o,
                  rowi, coli, bufu, dis2v, disv, c2v, c3v, acc,
                  semu, semr, semc):
    s = lax.axis_index("s")
    base = s * RPT
    pltpu.sync_copy(dis2r.at[pl.ds(s * (RPT // 8), RPT // 8)], dis2v)
    pltpu.sync_copy(disr.at[pl.ds(s * (RPT // 8), RPT // 8)], disv)
    pltpu.sync_copy(c2r, c2v)
    pltpu.sync_copy(c3r, c3v)

    srcs_u = [u0, u1o, u2o]
    dsts_u = [u1o, u2o, u3o]

    def idx_copy(j, slot):
        pltpu.make_async_copy(row3_hbm.at[s, j], rowi.at[slot], semr).start()
        pltpu.make_async_copy(col3_hbm.at[s, j], coli.at[slot], semc).start()

    def idx_wait(slot):
        pltpu.make_async_copy(row3_hbm.at[s, 0], rowi.at[slot], semr).wait()
        pltpu.make_async_copy(col3_hbm.at[s, 0], coli.at[slot], semc).wait()

    for k in range(3):
        su, du = srcs_u[k], dsts_u[k]
        for c in range(RPT // CHUNK):
            off = base + c * CHUNK
            pltpu.sync_copy(su.at[pl.ds(off, CHUNK)], bufu.at[0])
            pltpu.sync_copy(bufu.at[0], acc.at[pl.ds(GUARDU + off, CHUNK)])
        plsc.subcore_barrier()

        idx_copy(0, 0)
        idx_wait(0)
        idx_copy(1, 1)
        pltpu.make_async_copy(su.at[coli.at[0]], bufu.at[0], semu).start()

        def body(j, carry):
            slot = lax.rem(j, 2)
            nslot = 1 - slot
            pltpu.make_async_copy(su.at[coli.at[slot]], bufu.at[slot],
                                  semu).wait()

            @pl.when(j < NCHUNK - 1)
            def _():
                idx_wait(nslot)
                pltpu.make_async_copy(su.at[coli.at[nslot]],
                                      bufu.at[nslot], semu).start()

            pltpu.sync_copy(bufu.at[slot], acc.at[rowi.at[slot]], add=True)

            @pl.when(j < NCHUNK - 2)
            def _():
                idx_copy(j + 2, slot)

            return carry

        lax.fori_loop(0, NCHUNK, body, 0)
        plsc.subcore_barrier()

        cin = [c2v, c3v, None][k]
        for c in range(RPT // CHUNK):
            off = base + c * CHUNK
            pltpu.sync_copy(acc.at[pl.ds(GUARDU + off, CHUNK)], bufu.at[0])

            def ew(g, carry):
                for r8 in range(8):
                    dvec = dis2v[c * (CHUNK // 8) + g, pl.ds(16 * r8, 16)]
                    if cin is not None:
                        svec = disv[c * (CHUNK // 8) + g, pl.ds(16 * r8, 16)]
                    for l in range(D // 16):
                        val = bufu[0, g * 8 + r8, pl.ds(16 * l, 16)] * dvec
                        if cin is not None:
                            val = val + svec * cin[0, pl.ds(16 * l, 16)]
                        bufu[0, g * 8 + r8, pl.ds(16 * l, 16)] = val
                return carry

            lax.fori_loop(0, CHUNK // 8, ew, 0)
            pltpu.sync_copy(bufu.at[0], du.at[pl.ds(off, CHUNK)])
        plsc.subcore_barrier()


def _prep_body(w1t_ref, w2t_ref, w3t_ref, b1_ref, b2_ref,
               q_ref, c1_ref, c2_ref):
    t23 = jnp.dot(w2t_ref[...], w3t_ref[...],
                  preferred_element_type=jnp.float32)
    q_ref[...] = jnp.dot(w1t_ref[...], t23,
                         preferred_element_type=jnp.float32)
    c1_ref[...] = jnp.dot(b1_ref[...], t23,
                          preferred_element_type=jnp.float32)
    c2_ref[...] = jnp.dot(b2_ref[...], w3t_ref[...],
                          preferred_element_type=jnp.float32)


def _stage1_body(x_ref, q_ref, c1_ref, dp_ref, u0_ref, dis_ref, d2_ref):
    deg = dp_ref[:, 0:1] + 1.0
    dis = lax.rsqrt(deg)
    xq = jnp.dot(x_ref[...], q_ref[...], preferred_element_type=jnp.float32)
    u0_ref[...] = jnp.broadcast_to(dis, (BLK, D)) * (xq + c1_ref[...])
    dis_ref[...] = jnp.broadcast_to(dis, (BLK, 16))
    d2_ref[...] = jnp.broadcast_to(1.0 / deg, (BLK, 16))


def _final_body(u3_ref, dp_ref, out_ref):
    sq = jnp.sqrt(dp_ref[:, 0:1] + 1.0)
    out_ref[...] = sq * u3_ref[...]


_ROWS = pl.BlockSpec((BLK, D), lambda i: (i, 0))
_WMAT = pl.BlockSpec((D, D), lambda i: (0, 0))
_BVEC = pl.BlockSpec((1, D), lambda i: (0, 0))
_N16 = pl.BlockSpec((BLK, 16), lambda i: (i, 0))
_GRID = (N_PAD // BLK,)
_F32 = functools.partial(jax.ShapeDtypeStruct, dtype=jnp.float32)


def _prep(w1t, w2t, w3t, b1r, b2r):
    return pl.pallas_call(
        _prep_body,
        out_shape=[_F32((D, D)), _F32((1, D)), _F32((1, D))],
    )(w1t, w2t, w3t, b1r, b2r)


def _stage1(x_pad, q, c1, degp):
    return pl.pallas_call(
        _stage1_body,
        grid=_GRID,
        in_specs=[_ROWS, _WMAT, _BVEC, _ROWS],
        out_specs=[_ROWS, _N16, _N16],
        out_shape=[_F32((N_PAD, D)), _F32((N_PAD, 16)), _F32((N_PAD, 16))],
    )(x_pad, q, c1, degp)


def _final(u3, degp):
    return pl.pallas_call(
        _final_body,
        grid=_GRID,
        in_specs=[_ROWS, _ROWS],
        out_specs=_ROWS,
        out_shape=_F32((N_PAD, D)),
    )(u3, degp)


def kernel(x, edge_index, W1, b1, W2, b2, W3, b3):
    N = x.shape[0]
    E = edge_index.shape[1]
    row = edge_index[0].astype(jnp.int32)
    col = edge_index[1].astype(jnp.int32)
    dummy = jnp.full((E_PAD - E,), N, jnp.int32)
    row_p = jnp.concatenate([row, dummy])
    col_p = jnp.concatenate([col, dummy])
    row3 = ((row_p + GUARDU) * SC128).reshape(NS, NCHUNK, CHUNK)
    col3 = col_p.reshape(NS, NCHUNK, CHUNK)

    x_pad = jnp.zeros((N_PAD, D), jnp.float32).at[:N].set(x)
    zeros128 = jnp.zeros((N_PAD, D), jnp.float32)
    ones128 = jnp.ones((CHUNK, D), jnp.float32)
    w1t, w2t, w3t = W1.T, W2.T, W3.T
    b1r, b2r, b3r = b1.reshape(1, D), b2.reshape(1, D), b3.reshape(1, D)

    degp = _deg_kernel(col3, zeros128, ones128)
    q, c1, c2 = _prep(w1t, w2t, w3t, b1r, b2r)
    u0, disb, dis2b = _stage1(x_pad, q, c1, degp)
    dis2r = dis2b.reshape(N_PAD // 8, 8 * 16)
    disr = disb.reshape(N_PAD // 8, 8 * 16)
    _, _, u3 = _umega_kernel(u0, dis2r, disr, c2, b3r, row3, col3)
    h = _final(u3, degp)
    return h[:N]
